# Initial kernel scaffold; baseline (speedup 1.0000x reference)
import jax
import jax.numpy as jnp
from jax.experimental import pallas as pl


def kernel(input_labels, pos_labels, neg_labels, in_embed_weight, out_embed_weight):
    raise NotImplementedError("write your pallas kernel here")


# speedup vs baseline: 10.1435x; 10.1435x over previous
import functools

import jax
import jax.numpy as jnp
from jax import lax
from jax.experimental import pallas as pl
from jax.experimental.pallas import tpu as pltpu
from jax.experimental.pallas import tpu_sc as plsc

D = 64
B = 16384
CP = 20
NC, NS = 2, 16
NW = NC * NS

IN_PER_W = B // NW
POS_PER_W = B * CP // NW
POS_CHUNK = 1024
N_POS_CHUNKS = POS_PER_W // POS_CHUNK

_mesh = plsc.VectorSubcoreMesh(core_axis_name="c", subcore_axis_name="s")


@functools.partial(
    pl.kernel,
    mesh=_mesh,
    compiler_params=pltpu.CompilerParams(use_tc_tiling_on_sc=False),
    out_type=[
        jax.ShapeDtypeStruct((B, D), jnp.float32),
        jax.ShapeDtypeStruct((B * CP, D), jnp.float32),
    ],
    scratch_types=[
        pltpu.VMEM((IN_PER_W,), jnp.int32),
        pltpu.VMEM((POS_CHUNK,), jnp.int32),
        pltpu.VMEM((IN_PER_W, D), jnp.float32),
        pltpu.VMEM((POS_CHUNK, D), jnp.float32),
        pltpu.SemaphoreType.DMA,
    ],
)
def _sc_gather(inp_hbm, posflat_hbm, in_w_hbm, out_w_hbm,
               in_e_hbm, pos_e_hbm,
               idx_in_v, idx_pos_v, in_rows_v, pos_rows_v, sem):
    wid = lax.axis_index("s") * NC + lax.axis_index("c")

    in_base = wid * IN_PER_W
    pltpu.sync_copy(inp_hbm.at[pl.ds(in_base, IN_PER_W)], idx_in_v)
    pltpu.async_copy(in_w_hbm.at[idx_in_v], in_rows_v, sem).wait()
    pltpu.sync_copy(in_rows_v, in_e_hbm.at[pl.ds(in_base, IN_PER_W)])

    pos_base = wid * POS_PER_W

    @pl.loop(0, N_POS_CHUNKS)
    def _(t):
        base = pos_base + t * POS_CHUNK
        pltpu.sync_copy(posflat_hbm.at[pl.ds(base, POS_CHUNK)], idx_pos_v)
        pltpu.async_copy(out_w_hbm.at[idx_pos_v], pos_rows_v, sem).wait()
        pltpu.sync_copy(pos_rows_v, pos_e_hbm.at[pl.ds(base, POS_CHUNK)])


BB = 1024


def _tc_body(in_e_ref, pos_e_ref, out_ref):
    pe = pos_e_ref[...].reshape(BB, CP, D)
    ie = in_e_ref[...].reshape(BB, 1, D)
    d = jnp.sum(pe * ie, axis=2)
    f = jax.nn.softplus(d) + jax.nn.softplus(-d)
    out_ref[...] = jnp.sum(f, axis=1).reshape(1, BB)


_tc_compute = pl.pallas_call(
    _tc_body,
    grid=(B // BB,),
    in_specs=[
        pl.BlockSpec((BB, D), lambda i: (i, 0)),
        pl.BlockSpec((BB * CP, D), lambda i: (i, 0)),
    ],
    out_specs=pl.BlockSpec((1, BB), lambda i: (0, i)),
    out_shape=jax.ShapeDtypeStruct((1, B), jnp.float32),
)


def kernel(input_labels, pos_labels, neg_labels, in_embed_weight, out_embed_weight):
    del neg_labels
    inp = input_labels.astype(jnp.int32)
    pos_flat = pos_labels.astype(jnp.int32).reshape(B * CP)
    in_e, pos_e = _sc_gather(inp, pos_flat, in_embed_weight, out_embed_weight)
    out = _tc_compute(in_e, pos_e)
    return out.reshape(B)


# speedup vs baseline: 12.1995x; 1.2027x over previous
import functools

import jax
import jax.numpy as jnp
from jax import lax
from jax.experimental import pallas as pl
from jax.experimental.pallas import tpu as pltpu
from jax.experimental.pallas import tpu_sc as plsc

D = 64
L = 16
B = 16384
CP = 20
NC, NS = 2, 16
NW = NC * NS

B_PER_W = B // NW
CB = 32
N_CHUNKS = B_PER_W // CB
PPC = CB * CP

_mesh = plsc.VectorSubcoreMesh(core_axis_name="c", subcore_axis_name="s")


@functools.partial(
    pl.kernel,
    mesh=_mesh,
    compiler_params=pltpu.CompilerParams(use_tc_tiling_on_sc=False),
    out_type=jax.ShapeDtypeStruct((B * CP, L), jnp.float32),
    scratch_types=[
        pltpu.VMEM((CB,), jnp.int32),
        pltpu.VMEM((PPC,), jnp.int32),
        pltpu.VMEM((CB, D), jnp.float32),
        pltpu.VMEM((PPC, D), jnp.float32),
        pltpu.VMEM((PPC, L), jnp.float32),
        pltpu.SemaphoreType.DMA,
        pltpu.SemaphoreType.DMA,
    ],
)
def _sc_gather_dot(inp_hbm, posflat_hbm, in_w_hbm, out_w_hbm, q_hbm,
                   idx_in_v, idx_pos_v, in_rows_v, pos_rows_v, q_v,
                   sem_a, sem_b):
    wid = lax.axis_index("s") * NC + lax.axis_index("c")

    @pl.loop(0, N_CHUNKS)
    def _(t):
        base_b = wid * B_PER_W + t * CB
        base_p = base_b * CP
        pltpu.sync_copy(inp_hbm.at[pl.ds(base_b, CB)], idx_in_v)
        pltpu.sync_copy(posflat_hbm.at[pl.ds(base_p, PPC)], idx_pos_v)
        cp_in = pltpu.async_copy(in_w_hbm.at[idx_in_v], in_rows_v, sem_a)
        cp_pos = pltpu.async_copy(out_w_hbm.at[idx_pos_v], pos_rows_v, sem_b)
        cp_in.wait()
        cp_pos.wait()

        @pl.loop(0, CB)
        def _(i):
            a0 = in_rows_v[i, pl.ds(0, L)]
            a1 = in_rows_v[i, pl.ds(L, L)]
            a2 = in_rows_v[i, pl.ds(2 * L, L)]
            a3 = in_rows_v[i, pl.ds(3 * L, L)]
            row0 = i * CP
            for c in range(CP):
                q = (a0 * pos_rows_v[row0 + c, pl.ds(0, L)]
                     + a1 * pos_rows_v[row0 + c, pl.ds(L, L)]
                     + a2 * pos_rows_v[row0 + c, pl.ds(2 * L, L)]
                     + a3 * pos_rows_v[row0 + c, pl.ds(3 * L, L)])
                q_v[row0 + c, pl.ds(0, L)] = q

        pltpu.sync_copy(q_v, q_hbm.at[pl.ds(base_p, PPC)])


BB = 2048


def _tc_body(q_ref, out_ref):
    x = q_ref[...]
    seg = (jax.lax.broadcasted_iota(jnp.int32, (CP * L, CP), 0) // L
           == jax.lax.broadcasted_iota(jnp.int32, (CP * L, CP), 1))
    d = jax.lax.dot(x, seg.astype(jnp.float32),
                    precision=jax.lax.Precision.HIGHEST)
    f = jax.nn.softplus(d) + jax.nn.softplus(-d)
    out_ref[...] = jnp.sum(f, axis=1).reshape(1, BB)


_tc_compute = pl.pallas_call(
    _tc_body,
    grid=(B // BB,),
    in_specs=[pl.BlockSpec((BB, CP * L), lambda i: (i, 0))],
    out_specs=pl.BlockSpec((1, BB), lambda i: (0, i)),
    out_shape=jax.ShapeDtypeStruct((1, B), jnp.float32),
)


def kernel(input_labels, pos_labels, neg_labels, in_embed_weight, out_embed_weight):
    del neg_labels
    inp = input_labels.astype(jnp.int32)
    pos_flat = pos_labels.astype(jnp.int32).reshape(B * CP)
    q = _sc_gather_dot(inp, pos_flat, in_embed_weight, out_embed_weight)
    out = _tc_compute(q.reshape(B, CP * L))
    return out.reshape(B)
